# baseline (speedup 1.0000x reference)
import jax
import jax.numpy as jnp
import numpy as np
from jax.experimental import pallas as pl
from jax.experimental.pallas import tpu as pltpu

R = 1024
G = R // 16
N1 = 16384
STEPS = N1 // R

_S_SEL = (np.repeat(np.eye(G, dtype=np.float32), 16, axis=1) / 16.0).astype(
    np.float32)


def _lrelu(x):
    return jnp.where(x > 0, x, 0.01 * x)


def _sage_kernel(x2a_ref, x2b_ref, x1_ref, x0_ref, S_ref, Wn0_ref, Ws0_ref,
                 Wn1_ref, Ws1_ref, out_ref, b_acc, a0_acc):
    i = pl.program_id(0)
    Wn0 = Wn0_ref[...]
    Ws0 = Ws0_ref[...]
    S = S_ref[...]

    x2b = jnp.concatenate([x2a_ref[...], x2b_ref[...]], axis=0)
    a1 = jnp.mean(x2b.reshape(R, 16, 256), axis=1)
    x1b = x1_ref[...]
    h1 = _lrelu(
        jnp.dot(x1b, Ws0, preferred_element_type=jnp.float32)
        + jnp.dot(a1, Wn0, preferred_element_type=jnp.float32))
    b_acc[pl.ds(i * G, G), :] = jnp.dot(S, h1,
                                        preferred_element_type=jnp.float32)
    a0_acc[pl.ds(i * G, G), :] = jnp.dot(S, x1b,
                                         preferred_element_type=jnp.float32)

    @pl.when(i == STEPS - 1)
    def _final():
        x0 = x0_ref[...]
        h0 = _lrelu(
            jnp.dot(x0, Ws0, preferred_element_type=jnp.float32)
            + jnp.dot(a0_acc[...], Wn0, preferred_element_type=jnp.float32))
        out_ref[...] = (
            jnp.dot(h0, Ws1_ref[...], preferred_element_type=jnp.float32)
            + jnp.dot(b_acc[...], Wn1_ref[...],
                      preferred_element_type=jnp.float32))


def kernel(x0, x1, x2, Wn0, Ws0, Wn1, Ws1):
    return pl.pallas_call(
        _sage_kernel,
        grid=(STEPS,),
        in_specs=[
            pl.BlockSpec((R * 8, 256), lambda i: (2 * i, 0)),
            pl.BlockSpec((R * 8, 256), lambda i: (2 * i + 1, 0)),
            pl.BlockSpec((R, 256), lambda i: (i, 0)),
            pl.BlockSpec((1024, 256), lambda i: (0, 0)),
            pl.BlockSpec((G, R), lambda i: (0, 0)),
            pl.BlockSpec((256, 256), lambda i: (0, 0)),
            pl.BlockSpec((256, 256), lambda i: (0, 0)),
            pl.BlockSpec((256, 128), lambda i: (0, 0)),
            pl.BlockSpec((256, 128), lambda i: (0, 0)),
        ],
        out_specs=pl.BlockSpec((1024, 128), lambda i: (0, 0)),
        out_shape=jax.ShapeDtypeStruct((1024, 128), jnp.float32),
        scratch_shapes=[
            pltpu.VMEM((1024, 256), jnp.float32),
            pltpu.VMEM((1024, 256), jnp.float32),
        ],
    )(x2, x2, x1, x0, _S_SEL, Wn0, Ws0, Wn1, Ws1)


# speedup vs baseline: 1.0163x; 1.0163x over previous
import jax
import jax.numpy as jnp
import numpy as np
from jax.experimental import pallas as pl
from jax.experimental.pallas import tpu as pltpu

R = 1024
G = R // 16
N1 = 16384
STEPS = N1 // R

_S_SEL = (np.repeat(np.eye(G, dtype=np.float32), 16, axis=1) / 16.0).astype(
    np.float32)


def _lrelu(x):
    return jnp.where(x > 0, x, 0.01 * x)


def _sage_kernel(x2_ref, x1_ref, x0_ref, S_ref, Wn0_ref, Ws0_ref, Wn1_ref,
                 Ws1_ref, out_ref, b_acc, a0_acc):
    i = pl.program_id(0)
    Wn0 = Wn0_ref[...]
    Ws0 = Ws0_ref[...]
    S = S_ref[...]

    x2b = x2_ref[...]
    a1 = jnp.mean(x2b.reshape(R, 16, 256), axis=1)
    x1b = x1_ref[...]
    h1 = _lrelu(
        jnp.dot(x1b, Ws0, preferred_element_type=jnp.float32)
        + jnp.dot(a1, Wn0, preferred_element_type=jnp.float32))
    b_acc[pl.ds(i * G, G), :] = jnp.dot(S, h1,
                                        preferred_element_type=jnp.float32)
    a0_acc[pl.ds(i * G, G), :] = jnp.dot(S, x1b,
                                         preferred_element_type=jnp.float32)

    @pl.when(i == STEPS - 1)
    def _final():
        x0 = x0_ref[...]
        h0 = _lrelu(
            jnp.dot(x0, Ws0, preferred_element_type=jnp.float32)
            + jnp.dot(a0_acc[...], Wn0, preferred_element_type=jnp.float32))
        out_ref[...] = (
            jnp.dot(h0, Ws1_ref[...], preferred_element_type=jnp.float32)
            + jnp.dot(b_acc[...], Wn1_ref[...],
                      preferred_element_type=jnp.float32))


def kernel(x0, x1, x2, Wn0, Ws0, Wn1, Ws1):
    return pl.pallas_call(
        _sage_kernel,
        grid=(STEPS,),
        in_specs=[
            pl.BlockSpec((R * 16, 256), lambda i: (i, 0)),
            pl.BlockSpec((R, 256), lambda i: (i, 0)),
            pl.BlockSpec((1024, 256), lambda i: (0, 0)),
            pl.BlockSpec((G, R), lambda i: (0, 0)),
            pl.BlockSpec((256, 256), lambda i: (0, 0)),
            pl.BlockSpec((256, 256), lambda i: (0, 0)),
            pl.BlockSpec((256, 128), lambda i: (0, 0)),
            pl.BlockSpec((256, 128), lambda i: (0, 0)),
        ],
        out_specs=pl.BlockSpec((1024, 128), lambda i: (0, 0)),
        out_shape=jax.ShapeDtypeStruct((1024, 128), jnp.float32),
        scratch_shapes=[
            pltpu.VMEM((1024, 256), jnp.float32),
            pltpu.VMEM((1024, 256), jnp.float32),
        ],
    )(x2, x1, x0, _S_SEL, Wn0, Ws0, Wn1, Ws1)
